# baseline (speedup 1.0000x reference)
import functools

import jax
import jax.numpy as jnp
from jax import lax
from jax.experimental import pallas as pl
from jax.experimental.pallas import tpu as pltpu
from jax.experimental.pallas import tpu_sc as plsc

NUM_ITERS = 3
NC = 2
NS = 16
NW = NC * NS
LANES = 16
MASK16 = 0xFFFF
CAP = 8192
STRIDE = CAP // LANES


def _argmax_body(x_ref, o_ref):
    o_ref[...] = lax.argmax(x_ref[...], 1, jnp.int32)[:, None]


def _edge_step(np_, band, lo, lab_v, acc_v, col_v, row_v, wl_v, i):
    i32 = jnp.int32
    sl = pl.ds(i * i32(LANES), LANES)
    c = col_v[sl]
    r = row_v[sl]
    g0 = plsc.load_gather(lab_v, [c])
    g1 = plsc.load_gather(lab_v, [c + i32(np_)])
    a0 = g0 & MASK16
    a1 = (g0 >> 16) & MASK16
    a2 = g1 & MASK16
    a3 = (g1 >> 16) & MASK16
    w01 = wl_v[sl]
    w23 = wl_v[pl.ds(i32(CAP) + i * i32(LANES), LANES)]
    b0 = w01 & MASK16
    b1 = (w01 >> 16) & MASK16
    b2 = w23 & MASK16
    b3 = (w23 >> 16) & MASK16

    m00 = a0 * b0
    m01 = a0 * b1
    m10 = a1 * b0
    m02 = a0 * b2
    m11 = a1 * b1
    m20 = a2 * b0
    t = m00 & MASK16
    q0 = t
    p1 = ((m00 >> 16) & MASK16) + (m01 & MASK16) + (m10 & MASK16)
    t = p1 + (t >> 16)
    q1 = t & MASK16
    p2 = (((m01 >> 16) & MASK16) + ((m10 >> 16) & MASK16)
          + (m02 & MASK16) + (m11 & MASK16) + (m20 & MASK16))
    t = p2 + (t >> 16)
    q2 = t & MASK16
    p3 = (a0 * b3 + a1 * b2 + a2 * b1 + a3 * b0
          + (m02 >> 16) + (m11 >> 16) + (m20 >> 16))
    t = p3 + (t >> 16)
    q3 = t & MASK16

    msk = (r >= lo) & (r < lo + i32(band))
    ridx = jnp.clip(r - lo, 0, band - 1)
    plsc.addupdate_scatter(acc_v, [ridx], q0, mask=msk)
    plsc.addupdate_scatter(acc_v, [ridx + i32(band)], q1, mask=msk)
    plsc.addupdate_scatter(acc_v, [ridx + i32(2 * band)], q2, mask=msk)
    plsc.addupdate_scatter(acc_v, [ridx + i32(3 * band)], q3, mask=msk)


def _scatter_body(np_, labels_hbm, col_hbm, row_hbm, wl_hbm,
                  out_hbm, lab_v, acc_v, col_v, row_v, wl_v):
    i32 = jnp.int32
    band = np_ // NW
    wid = lax.axis_index("s") * i32(NC) + lax.axis_index("c")
    base = pl.multiple_of(wid * i32(CAP), 8)
    lo = pl.multiple_of(wid * i32(band), 8)
    pltpu.sync_copy(labels_hbm, lab_v)
    pltpu.sync_copy(col_hbm.at[pl.ds(base, CAP)], col_v)
    pltpu.sync_copy(row_hbm.at[pl.ds(base, CAP)], row_v)
    for l in range(2):
        pltpu.sync_copy(wl_hbm.at[pl.ds(i32(l * NW * CAP) + base, CAP)],
                        wl_v.at[pl.ds(i32(l * CAP), CAP)])

    zeros = jnp.zeros((LANES,), jnp.int32)

    def zero_body(i, carry):
        acc_v[pl.ds(i * i32(LANES), LANES)] = zeros
        return carry

    lax.fori_loop(i32(0), i32((4 * band) // LANES), zero_body, i32(0))

    EU = 4
    step = functools.partial(_edge_step, np_, band, lo, lab_v, acc_v, col_v,
                             row_v, wl_v)

    def edge_body(i, carry):
        for u in range(EU):
            step(i * i32(EU) + i32(u))
        return carry

    lax.fori_loop(i32(0), i32(CAP // (EU * LANES)), edge_body, i32(0))

    def norm_body(i, carry):
        off = i * i32(LANES)
        sls = [pl.ds(i32(l * band) + off, LANES) for l in range(4)]
        t0 = acc_v[sls[0]]
        q0 = t0 & MASK16
        t1 = acc_v[sls[1]] + (t0 >> 16)
        q1 = t1 & MASK16
        t2 = acc_v[sls[2]] + (t1 >> 16)
        q2 = t2 & MASK16
        t3 = acc_v[sls[3]] + (t2 >> 16)
        q3 = t3 & MASK16
        acc_v[sls[0]] = q0 | (q1 << 16)
        acc_v[sls[1]] = q2 | (q3 << 16)
        return carry

    lax.fori_loop(i32(0), i32(band // LANES), norm_body, i32(0))

    pltpu.sync_copy(acc_v.at[pl.ds(i32(0), band)],
                    out_hbm.at[pl.ds(lo, band)])
    pltpu.sync_copy(acc_v.at[pl.ds(i32(band), band)],
                    out_hbm.at[pl.ds(pl.multiple_of(i32(np_) + lo, 8), band)])


def _pack_to_i64(pack, n):
    lo = pack[:n]
    hi = pack[pack.shape[0] // 2:][:n]
    return (hi.astype(jnp.int64) << 32) | lo.astype(jnp.uint32).astype(jnp.int64)


def kernel(x, edge_index):
    n_nodes, _ = x.shape
    n_edges = edge_index.shape[1]

    np_ = ((n_nodes + 1 + 255) // 256) * 256
    band = np_ // NW

    row = edge_index[0].astype(jnp.int32)
    col = edge_index[1].astype(jnp.int32)

    idx = jnp.arange(n_edges, dtype=jnp.int32)
    key = (row.astype(jnp.uint32) << 18) | idx.astype(jnp.uint32)
    key_s = jnp.sort(key)
    row_s = (key_s >> 18).astype(jnp.int32)
    perm = (key_s & jnp.uint32((1 << 18) - 1)).astype(jnp.int32)
    col_s = col[perm]

    is_last = jnp.concatenate([row_s[1:] != row_s[:-1],
                               jnp.ones((1,), dtype=bool)])
    endv = jnp.where(is_last, idx, jnp.int32(n_edges))
    end_idx = jnp.flip(lax.cummin(jnp.flip(endv)))
    k_cnt = (end_idx - idx).astype(jnp.uint64)
    w = jnp.ones((n_edges,), jnp.uint64)
    basep = jnp.uint64(31)
    for b in range(18):
        bit = (k_cnt >> jnp.uint64(b)) & jnp.uint64(1)
        w = jnp.where(bit == jnp.uint64(1), w * basep, w)
        basep = basep * basep
    wlo = lax.bitcast_convert_type((w & jnp.uint64(0xFFFFFFFF)).astype(jnp.uint32),
                                   jnp.int32)
    whi = lax.bitcast_convert_type((w >> jnp.uint64(32)).astype(jnp.uint32),
                                   jnp.int32)

    bands = jnp.arange(NW, dtype=jnp.int32) * band
    start = jnp.searchsorted(row_s, bands, side="left").astype(jnp.int32)
    a = (start // 16) * 16
    i_slot = jnp.arange(CAP, dtype=jnp.int32)
    src_off = (i_slot // LANES) + (i_slot % LANES) * STRIDE
    src = a[:, None] + src_off[None, :]
    valid = src < n_edges
    srcc = jnp.minimum(src, n_edges - 1)
    row_slot = jnp.where(valid, row_s[srcc], jnp.int32(np_)).reshape(-1)
    col_slot = jnp.where(valid, col_s[srcc], 0).reshape(-1)
    wlo_slot = jnp.where(valid, wlo[srcc], 0).reshape(-1)
    whi_slot = jnp.where(valid, whi[srcc], 0).reshape(-1)
    w_slot = jnp.concatenate([wlo_slot, whi_slot])

    x_pad = jnp.pad(x, ((0, np_ - n_nodes), (0, 0)))
    labels0 = pl.pallas_call(
        _argmax_body,
        out_shape=jax.ShapeDtypeStruct((np_, 1), jnp.int32),
    )(x_pad)[:, 0]
    labels_flat = jnp.zeros((2, np_), jnp.int32).at[0].set(labels0).reshape(-1)

    mesh = plsc.VectorSubcoreMesh(core_axis_name="c", subcore_axis_name="s",
                                  num_cores=NC, num_subcores=NS)
    scatter_k = functools.partial(
        pl.kernel,
        out_type=jax.ShapeDtypeStruct((2 * np_,), jnp.int32),
        mesh=mesh,
        compiler_params=pltpu.CompilerParams(needs_layout_passes=False),
        scratch_types=[
            pltpu.VMEM((2 * np_,), jnp.int32),
            pltpu.VMEM((4 * band,), jnp.int32),
            pltpu.VMEM((CAP,), jnp.int32),
            pltpu.VMEM((CAP,), jnp.int32),
            pltpu.VMEM((2 * CAP,), jnp.int32),
        ],
    )(functools.partial(_scatter_body, np_))

    history = [labels0[:n_nodes].astype(jnp.int64)]
    for _ in range(NUM_ITERS):
        labels_flat = scatter_k(labels_flat, col_slot, row_slot, w_slot)
        history.append(_pack_to_i64(labels_flat, n_nodes))

    return history[-1], jnp.stack(history)
